# baseline (speedup 1.0000x reference)
import functools

import jax
import jax.numpy as jnp
from jax import lax
from jax.experimental import pallas as pl
from jax.experimental.pallas import tpu as pltpu
from jax.experimental.pallas import tpu_sc as plsc

BATCH = 4096
SEQ = 200
D = 16
SMILES_V = 100000
PROTEIN_V = 1000000
NC = 2
NS = 16
NW = NC * NS
TT = SEQ // 8
CHUNK = 8 * 128

_mesh = plsc.VectorSubcoreMesh(core_axis_name="c", subcore_axis_name="s")

_SCRATCH = [
    pltpu.VMEM((2, CHUNK), jnp.int32),
    pltpu.VMEM((2, CHUNK, D), jnp.float32),
    pltpu.VMEM((8, 2, 8, 128), jnp.float32),
    pltpu.SemaphoreType.DMA((2,)),
    pltpu.SemaphoreType.DMA((2,)),
    pltpu.SemaphoreType.DMA((2,)),
]


def _embed_one(idx4, table, idx_v, rows_v, out_buf, sem_i, sem_g, sem_o,
               out5, wid, iota):
    def idx_copy(tt, b):
        return pltpu.make_async_copy(
            idx4.at[tt, wid], idx_v.at[b], sem_i.at[b])

    def gather(b):
        return pltpu.make_async_copy(
            table.at[idx_v.at[b]], rows_v.at[b], sem_g.at[b])

    def transpose_and_emit(tt, b):
        rv = rows_v.at[b]
        for ti in range(8):
            rows8 = [iota + (ti * 128 + 16 * g) for g in range(8)]
            for d in range(16):
                d_col = jnp.full((16,), d, jnp.int32)
                for g in range(8):
                    v = plsc.load_gather(rv, [rows8[g], d_col])
                    out_buf[ti, d // 8, d % 8, pl.ds(16 * g, 16)] = v
            pltpu.make_async_copy(
                out_buf.at[ti], out5.at[8 * tt + ti, :, wid],
                sem_o.at[b]).start()
        for ti in range(8):
            pltpu.make_async_copy(
                out_buf.at[ti], out5.at[8 * tt + ti, :, wid],
                sem_o.at[b]).wait()

    idx_copy(0, 0).start()
    idx_copy(1, 1).start()
    idx_copy(0, 0).wait()
    gather(0).start()

    def tile_body(tt, carry):
        b = lax.rem(tt, 2)
        nb = 1 - b

        @pl.when(tt < 24)
        def _():
            idx_copy(tt + 1, nb).wait()
            gather(nb).start()

        gather(b).wait()
        transpose_and_emit(tt, b)

        @pl.when(tt < 23)
        def _():
            idx_copy(tt + 2, b).start()

        return carry

    lax.fori_loop(0, 25, tile_body, 0)


@functools.partial(
    pl.kernel,
    out_type=jax.ShapeDtypeStruct((SEQ, 2, 32, 8, 128), jnp.float32),
    mesh=_mesh,
    compiler_params=pltpu.CompilerParams(
        use_tc_tiling_on_sc=False, needs_layout_passes=False),
    scratch_types=_SCRATCH,
)
def _embed_smiles(idx4, table, out5, idx_v, rows_v, out_buf,
                  sem_i, sem_g, sem_o):
    wid = lax.axis_index("s") * NC + lax.axis_index("c")
    iota = lax.iota(jnp.int32, 16)
    _embed_one(idx4, table, idx_v, rows_v, out_buf, sem_i, sem_g, sem_o,
               out5, wid, iota)


@functools.partial(
    pl.kernel,
    out_type=jax.ShapeDtypeStruct((SEQ, 2, 32, 8, 128), jnp.float32),
    mesh=_mesh,
    compiler_params=pltpu.CompilerParams(
        use_tc_tiling_on_sc=False, needs_layout_passes=False),
    scratch_types=_SCRATCH,
)
def _embed_protein(idx4, table, out5, idx_v, rows_v, out_buf,
                   sem_i, sem_g, sem_o):
    wid = lax.axis_index("s") * NC + lax.axis_index("c")
    iota = lax.iota(jnp.int32, 16)
    _embed_one(idx4, table, idx_v, rows_v, out_buf, sem_i, sem_g, sem_o,
               out5, wid, iota)


def kernel(smiles_data, protein_data, W_smiles, W_protein):
    s4 = smiles_data.reshape(32, 128, TT, 8).transpose(2, 0, 3, 1).reshape(TT, 32, CHUNK)
    p4 = protein_data.reshape(32, 128, TT, 8).transpose(2, 0, 3, 1).reshape(TT, 32, CHUNK)
    out_s5 = _embed_smiles(s4, W_smiles)
    out_p5 = _embed_protein(p4, W_protein)
    out_s = out_s5.transpose(2, 4, 0, 1, 3).reshape(BATCH, SEQ, D)
    out_p = out_p5.transpose(2, 4, 0, 1, 3).reshape(BATCH, SEQ, D)
    return (out_s, out_p)


# speedup vs baseline: 1.2362x; 1.2362x over previous
import functools

import jax
import jax.numpy as jnp
from jax import lax
from jax.experimental import pallas as pl
from jax.experimental.pallas import tpu as pltpu
from jax.experimental.pallas import tpu_sc as plsc

BATCH = 4096
SEQ = 200
D = 16
SMILES_V = 100000
PROTEIN_V = 1000000
NC = 2
NS = 16
NW = NC * NS
TT = SEQ // 8
CHUNK = 8 * 128

_mesh = plsc.VectorSubcoreMesh(core_axis_name="c", subcore_axis_name="s")

_SCRATCH = [
    pltpu.VMEM((2, CHUNK), jnp.int32),
    pltpu.VMEM((2, CHUNK, D), jnp.float32),
    pltpu.VMEM((8, 2, 8, 128), jnp.float32),
    pltpu.SemaphoreType.DMA((2,)),
    pltpu.SemaphoreType.DMA((2,)),
    pltpu.SemaphoreType.DMA((2,)),
]


def _embed_one(idx4, table, idx_v, rows_v, out_buf, sem_i, sem_g, sem_o,
               out5, wid, iota):
    def idx_copy(tt, b):
        return pltpu.make_async_copy(
            idx4.at[tt, wid], idx_v.at[b], sem_i.at[b])

    def gather(b):
        return pltpu.make_async_copy(
            table.at[idx_v.at[b]], rows_v.at[b], sem_g.at[b])

    def transpose_and_emit(tt, b):
        rv = rows_v.at[b]
        for ti in range(8):
            rows8 = [iota + (ti * 128 + 16 * g) for g in range(8)]

            @plsc.parallel_loop(0, 16, unroll=4)
            def _(d):
                d_col = jnp.full((16,), d, jnp.int32)
                for g in range(8):
                    v = plsc.load_gather(rv, [rows8[g], d_col])
                    dd = lax.div(d, 8)
                    dm = lax.rem(d, 8)
                    out_buf[ti, dd, dm, pl.ds(16 * g, 16)] = v

            pltpu.make_async_copy(
                out_buf.at[ti], out5.at[8 * tt + ti, :, wid],
                sem_o.at[b]).start()
        for ti in range(8):
            pltpu.make_async_copy(
                out_buf.at[ti], out5.at[8 * tt + ti, :, wid],
                sem_o.at[b]).wait()

    idx_copy(0, 0).start()
    idx_copy(1, 1).start()
    idx_copy(0, 0).wait()
    gather(0).start()

    def tile_body(tt, carry):
        b = lax.rem(tt, 2)
        nb = 1 - b

        @pl.when(tt < 24)
        def _():
            idx_copy(tt + 1, nb).wait()
            gather(nb).start()

        gather(b).wait()
        transpose_and_emit(tt, b)

        @pl.when(tt < 23)
        def _():
            idx_copy(tt + 2, b).start()

        return carry

    lax.fori_loop(0, 25, tile_body, 0)


@functools.partial(
    pl.kernel,
    out_type=jax.ShapeDtypeStruct((SEQ, 2, 32, 8, 128), jnp.float32),
    mesh=_mesh,
    compiler_params=pltpu.CompilerParams(
        use_tc_tiling_on_sc=False, needs_layout_passes=False),
    scratch_types=_SCRATCH,
)
def _embed_smiles(idx4, table, out5, idx_v, rows_v, out_buf,
                  sem_i, sem_g, sem_o):
    wid = lax.axis_index("s") * NC + lax.axis_index("c")
    iota = lax.iota(jnp.int32, 16)
    _embed_one(idx4, table, idx_v, rows_v, out_buf, sem_i, sem_g, sem_o,
               out5, wid, iota)


@functools.partial(
    pl.kernel,
    out_type=jax.ShapeDtypeStruct((SEQ, 2, 32, 8, 128), jnp.float32),
    mesh=_mesh,
    compiler_params=pltpu.CompilerParams(
        use_tc_tiling_on_sc=False, needs_layout_passes=False),
    scratch_types=_SCRATCH,
)
def _embed_protein(idx4, table, out5, idx_v, rows_v, out_buf,
                   sem_i, sem_g, sem_o):
    wid = lax.axis_index("s") * NC + lax.axis_index("c")
    iota = lax.iota(jnp.int32, 16)
    _embed_one(idx4, table, idx_v, rows_v, out_buf, sem_i, sem_g, sem_o,
               out5, wid, iota)


def kernel(smiles_data, protein_data, W_smiles, W_protein):
    s4 = smiles_data.reshape(32, 128, TT, 8).transpose(2, 0, 3, 1).reshape(TT, 32, CHUNK)
    p4 = protein_data.reshape(32, 128, TT, 8).transpose(2, 0, 3, 1).reshape(TT, 32, CHUNK)
    out_s5 = _embed_smiles(s4, W_smiles)
    out_p5 = _embed_protein(p4, W_protein)
    out_s = out_s5.transpose(2, 4, 0, 1, 3).reshape(BATCH, SEQ, D)
    out_p = out_p5.transpose(2, 4, 0, 1, 3).reshape(BATCH, SEQ, D)
    return (out_s, out_p)
